# Initial kernel scaffold; baseline (speedup 1.0000x reference)
import jax
import jax.numpy as jnp
from jax.experimental import pallas as pl


def kernel(x, edge_index, edge_attr, batch, params):
    raise NotImplementedError("write your pallas kernel here")


# speedup vs baseline: 19.6508x; 19.6508x over previous
import functools

import jax
import jax.numpy as jnp
import numpy as np
from jax import lax
from jax.experimental import pallas as pl
from jax.experimental.pallas import tpu as pltpu
from jax.experimental.pallas import tpu_sc as plsc

N = 10000
E = 320000
D = 128
H = 4
C = 32
NB = 16
OUT = 100
F32 = jnp.float32

NC = 2
NS = 16
NW = NC * NS
EPW = E // NW
CH = 80
NCHUNK = EPW // CH
NPAD = 10240
RPT = NPAD // NS

BN = 1000
BE = 2000

@functools.cache
def _sc_mesh():
    return plsc.VectorSubcoreMesh(core_axis_name="c", subcore_axis_name="s",
                                  num_cores=NC, num_subcores=NS)


def _sc_gather_body(qt, kvt, dst_h, src_h, gq, gkv,
                    idxd, idxs, qrows, kvrows, semq, semk):
    cid = lax.axis_index("c")
    sid = lax.axis_index("s")
    wid = sid * NC + cid

    def step(i, carry):
        base = wid * EPW + i * CH
        pltpu.sync_copy(dst_h.at[pl.ds(base, CH)], idxd)
        pltpu.sync_copy(src_h.at[pl.ds(base, CH)], idxs)
        cq = pltpu.async_copy(qt.at[idxd], qrows, semq)
        ck = pltpu.async_copy(kvt.at[idxs], kvrows, semk)
        cq.wait()
        ck.wait()
        pltpu.sync_copy(qrows, gq.at[pl.ds(base, CH)])
        pltpu.sync_copy(kvrows, gkv.at[pl.ds(base, CH)])
        return carry

    lax.fori_loop(0, NCHUNK, step, 0)


def _sc_gather(qt, kvt, dst, src):
    return pl.kernel(
        _sc_gather_body,
        out_type=(jax.ShapeDtypeStruct((E, D), F32),
                  jax.ShapeDtypeStruct((E, 2 * D), F32)),
        mesh=_sc_mesh(),
        scratch_types=[
            pltpu.VMEM((CH,), jnp.int32),
            pltpu.VMEM((CH,), jnp.int32),
            pltpu.VMEM((CH, D), F32),
            pltpu.VMEM((CH, 2 * D), F32),
            pltpu.SemaphoreType.DMA,
            pltpu.SemaphoreType.DMA,
        ],
    )(qt, kvt, dst, src)


def _sc_scatter_body(wv_h, pb_h, dst_h, zrow_h, op_h, sp_h,
                     idx, rows, acc):
    cid = lax.axis_index("c")
    sid = lax.axis_index("s")
    wid = sid * NC + cid

    for src_h, out_h in ((wv_h, op_h), (pb_h, sp_h)):
        pltpu.sync_copy(zrow_h, acc.at[pl.ds(sid * RPT, RPT)])
        plsc.subcore_barrier()

        def step(i, carry, src_h=src_h):
            base = wid * EPW + i * CH
            pltpu.sync_copy(dst_h.at[pl.ds(base, CH)], idx)
            pltpu.sync_copy(src_h.at[pl.ds(base, CH)], rows)
            pltpu.sync_copy(rows, acc.at[idx], add=True)
            return carry

        lax.fori_loop(0, NCHUNK, step, 0)
        plsc.subcore_barrier()
        pltpu.sync_copy(acc.at[pl.ds(sid * RPT, RPT)],
                        out_h.at[cid, pl.ds(sid * RPT, RPT)])
        plsc.subcore_barrier()


def _sc_scatter(wv, pb, dst, zrow):
    return pl.kernel(
        _sc_scatter_body,
        out_type=(jax.ShapeDtypeStruct((NC, NPAD, D), F32),
                  jax.ShapeDtypeStruct((NC, NPAD, D), F32)),
        mesh=_sc_mesh(),
        scratch_types=[
            pltpu.VMEM((CH,), jnp.int32),
            pltpu.VMEM((CH, D), F32),
            pltpu.VMEM_SHARED((NPAD, D), F32),
        ],
    )(wv, pb, dst, zrow)


def _head_mats(dtype=F32):
    ci = lax.broadcasted_iota(jnp.int32, (D, 8), 0)
    hi = lax.broadcasted_iota(jnp.int32, (D, 8), 1)
    hd = jnp.where((hi < H) & (ci // C == hi), 1.0, 0.0).astype(dtype)
    hi2 = lax.broadcasted_iota(jnp.int32, (8, D), 0)
    ci2 = lax.broadcasted_iota(jnp.int32, (8, D), 1)
    hx = jnp.where((hi2 < H) & (ci2 // C == hi2), 1.0, 0.0).astype(dtype)
    return hd, hx


def _dense0_body(x_ref, w_ref, b_ref, h_ref):
    h_ref[...] = (jnp.dot(x_ref[...], w_ref[...],
                          preferred_element_type=F32) + b_ref[...])


def _dense0(x8, npw8, npb):
    return pl.pallas_call(
        _dense0_body,
        grid=(N // BN,),
        in_specs=[
            pl.BlockSpec((BN, 8), lambda i: (i, 0)),
            pl.BlockSpec((8, D), lambda i: (0, 0)),
            pl.BlockSpec((1, D), lambda i: (0, 0)),
        ],
        out_specs=pl.BlockSpec((BN, D), lambda i: (i, 0)),
        out_shape=jax.ShapeDtypeStruct((N, D), F32),
    )(x8, npw8, npb)


def _qkv_body(h_ref, w_ref, bq_ref, bk_ref, bv_ref, we_ref, epb_ref, be_ref,
              qt_ref, kvt_ref):
    hw = jnp.dot(h_ref[...], w_ref[...], preferred_element_type=F32)
    de = (jnp.dot(epb_ref[...], we_ref[...], preferred_element_type=F32)
          + be_ref[...])
    qt_ref[...] = hw[:, :D] + bq_ref[...]
    kvt_ref[...] = hw[:, D:] + jnp.concatenate(
        [bk_ref[...] + de, bv_ref[...] + de], axis=1)


def _qkv(h, w3, bq, bk, bv, we, epb, be):
    return pl.pallas_call(
        _qkv_body,
        grid=(N // BN,),
        in_specs=[
            pl.BlockSpec((BN, D), lambda i: (i, 0)),
            pl.BlockSpec((D, 3 * D), lambda i: (0, 0)),
            pl.BlockSpec((1, D), lambda i: (0, 0)),
            pl.BlockSpec((1, D), lambda i: (0, 0)),
            pl.BlockSpec((1, D), lambda i: (0, 0)),
            pl.BlockSpec((D, D), lambda i: (0, 0)),
            pl.BlockSpec((1, D), lambda i: (0, 0)),
            pl.BlockSpec((1, D), lambda i: (0, 0)),
        ],
        out_specs=(pl.BlockSpec((BN, D), lambda i: (i, 0)),
                   pl.BlockSpec((BN, 2 * D), lambda i: (i, 0))),
        out_shape=(jax.ShapeDtypeStruct((N, D), F32),
                   jax.ShapeDtypeStruct((N, 2 * D), F32)),
    )(h, w3, bq, bk, bv, we, epb, be)


def _ee_body(ea_ref, epw_ref, we_ref, ee_ref):
    ce = jnp.dot(epw_ref[...], we_ref[...], preferred_element_type=F32)
    ee_ref[...] = jnp.dot(ea_ref[...], ce, preferred_element_type=F32)


def _ee(ea8, epw8, we):
    return pl.pallas_call(
        _ee_body,
        grid=(E // BE,),
        in_specs=[
            pl.BlockSpec((BE, 8), lambda i: (i, 0)),
            pl.BlockSpec((8, D), lambda i: (0, 0)),
            pl.BlockSpec((D, D), lambda i: (0, 0)),
        ],
        out_specs=pl.BlockSpec((BE, D), lambda i: (i, 0)),
        out_shape=jax.ShapeDtypeStruct((E, D), F32),
    )(ea8, epw8, we)


def _alpha_body(gq_ref, gkv_ref, ee_ref, wv_ref, pb_ref):
    hd, hx = _head_mats()
    ee = ee_ref[...]
    kj = gkv_ref[:, :D] + ee
    vj = gkv_ref[:, D:] + ee
    prod = gq_ref[...] * kj
    alpha = jnp.dot(prod, hd, preferred_element_type=F32) * (1.0 / np.sqrt(C))
    p = jnp.exp(alpha)
    pbig = jnp.dot(p, hx, preferred_element_type=F32)
    wv_ref[...] = vj * pbig
    pb_ref[...] = pbig


def _alpha(gq, gkv, ee):
    return pl.pallas_call(
        _alpha_body,
        grid=(E // BE,),
        in_specs=[
            pl.BlockSpec((BE, D), lambda i: (i, 0)),
            pl.BlockSpec((BE, 2 * D), lambda i: (i, 0)),
            pl.BlockSpec((BE, D), lambda i: (i, 0)),
        ],
        out_specs=(pl.BlockSpec((BE, D), lambda i: (i, 0)),
                   pl.BlockSpec((BE, D), lambda i: (i, 0))),
        out_shape=(jax.ShapeDtypeStruct((E, D), F32),
                   jax.ShapeDtypeStruct((E, D), F32)),
    )(gq, gkv, ee)


def _epi_body(op_ref, sp_ref, h_ref, ws_ref, bs_ref, g_ref, b_ref, hn_ref):
    o = op_ref[0] + op_ref[1]
    sb = sp_ref[0] + sp_ref[1]
    out = o / (sb + 1e-16)
    out = out + jnp.dot(h_ref[...], ws_ref[...],
                        preferred_element_type=F32) + bs_ref[...]
    mu = jnp.mean(out, axis=1, keepdims=True)
    var = jnp.mean((out - mu) ** 2, axis=1, keepdims=True)
    out = (out - mu) * lax.rsqrt(var + 1e-5) * g_ref[...] + b_ref[...]
    hn_ref[...] = jnp.maximum(out, 0.0)


def _epi(op, sp, h, ws, bs, g, b):
    return pl.pallas_call(
        _epi_body,
        grid=(N // BN,),
        in_specs=[
            pl.BlockSpec((NC, BN, D), lambda i: (0, i, 0)),
            pl.BlockSpec((NC, BN, D), lambda i: (0, i, 0)),
            pl.BlockSpec((BN, D), lambda i: (i, 0)),
            pl.BlockSpec((D, D), lambda i: (0, 0)),
            pl.BlockSpec((1, D), lambda i: (0, 0)),
            pl.BlockSpec((1, D), lambda i: (0, 0)),
            pl.BlockSpec((1, D), lambda i: (0, 0)),
        ],
        out_specs=pl.BlockSpec((BN, D), lambda i: (i, 0)),
        out_shape=jax.ShapeDtypeStruct((N, D), F32),
    )(op, sp, h, ws, bs, g, b)


def _readout_body(h_ref, bb_ref, d1_ref, db1_ref, d2_ref, db2_ref,
                  t1_ref, tb1_ref, t2_ref, tb2_ref,
                  dos_ref, trans_ref, sums_ref, cnt_ref):
    i = pl.program_id(0)

    @pl.when(i == 0)
    def _init():
        sums_ref[...] = jnp.zeros_like(sums_ref)
        cnt_ref[...] = jnp.zeros_like(cnt_ref)

    hh = h_ref[...]
    bb = bb_ref[...][:, 0:1]
    ids = lax.broadcasted_iota(jnp.int32, (BN, NB), 1).astype(F32)
    onehot = jnp.where(jnp.broadcast_to(bb, (BN, NB)) == ids, 1.0, 0.0)
    dn = (((0,), (0,)), ((), ()))
    sums_ref[...] += lax.dot_general(onehot, hh, dn,
                                     preferred_element_type=F32)
    cnt_ref[...] += lax.dot_general(onehot, jnp.ones_like(hh), dn,
                                    preferred_element_type=F32)

    @pl.when(i == (N // BN) - 1)
    def _fin():
        g = sums_ref[...] / jnp.maximum(cnt_ref[...], 1.0)
        dd = jnp.maximum(
            jnp.dot(g, d1_ref[...], preferred_element_type=F32)
            + db1_ref[...], 0.0)
        dos_ref[...] = (jnp.dot(dd, d2_ref[...], preferred_element_type=F32)
                        + db2_ref[...])
        tt = jnp.maximum(
            jnp.dot(g, t1_ref[...], preferred_element_type=F32)
            + tb1_ref[...], 0.0)
        trans_ref[...] = (jnp.dot(tt, t2_ref[...], preferred_element_type=F32)
                          + tb2_ref[...])


def _readout(h, bb8, d1, db1, d2, db2, t1, tb1, t2, tb2):
    D2 = D // 2
    return pl.pallas_call(
        _readout_body,
        grid=(N // BN,),
        in_specs=[
            pl.BlockSpec((BN, D), lambda i: (i, 0)),
            pl.BlockSpec((BN, 8), lambda i: (i, 0)),
            pl.BlockSpec((D, D2), lambda i: (0, 0)),
            pl.BlockSpec((1, D2), lambda i: (0, 0)),
            pl.BlockSpec((D2, OUT), lambda i: (0, 0)),
            pl.BlockSpec((1, OUT), lambda i: (0, 0)),
            pl.BlockSpec((D, D2), lambda i: (0, 0)),
            pl.BlockSpec((1, D2), lambda i: (0, 0)),
            pl.BlockSpec((D2, OUT), lambda i: (0, 0)),
            pl.BlockSpec((1, OUT), lambda i: (0, 0)),
        ],
        out_specs=(pl.BlockSpec((NB, OUT), lambda i: (0, 0)),
                   pl.BlockSpec((NB, OUT), lambda i: (0, 0))),
        out_shape=(jax.ShapeDtypeStruct((NB, OUT), F32),
                   jax.ShapeDtypeStruct((NB, OUT), F32)),
        scratch_shapes=[pltpu.VMEM((NB, D), F32), pltpu.VMEM((NB, D), F32)],
    )(h, bb8, d1, db1, d2, db2, t1, tb1, t2, tb2)


def _row(v):
    return v.reshape(1, -1)


def kernel(x, edge_index, edge_attr, batch, params):
    if False:
        src = edge_index[0]
        dst = edge_index[1]
        n = x.shape[0]
        h = x @ params['np_w'] + params['np_b']
        e = edge_attr @ params['ep_w'] + params['ep_b']
        for lp in params['layers']:
            q = (h @ lp['wq'] + lp['bq'])
            k = (h @ lp['wk'] + lp['bk'])
            v = (h @ lp['wv'] + lp['bv'])
            ee = (e @ lp['we'] + lp['be'])
            kj = k[src] + ee
            vj = v[src] + ee
            alpha = jnp.sum((q[dst] * kj).reshape(-1, H, C), axis=-1) / np.sqrt(C)
            p = jnp.exp(alpha)
            O = jax.ops.segment_sum(vj * jnp.repeat(p, C, axis=1), dst,
                                    num_segments=n)
            S = jax.ops.segment_sum(p, dst, num_segments=n)
            out = O / (jnp.repeat(S, C, axis=1) + 1e-16)
            out = out + h @ lp['ws'] + lp['bs']
            mu = jnp.mean(out, axis=-1, keepdims=True)
            var = jnp.var(out, axis=-1, keepdims=True)
            out = (out - mu) / jnp.sqrt(var + 1e-5) * lp['ln_g'] + lp['ln_b']
            h = jax.nn.relu(out)
        sums = jax.ops.segment_sum(h, batch, num_segments=NB)
        cnt = jax.ops.segment_sum(jnp.ones((n,), h.dtype), batch,
                                  num_segments=NB)
        g = sums / jnp.maximum(cnt, 1.0)[:, None]
        dos = (jax.nn.relu(g @ params['d1_w'] + params['d1_b'])
               @ params['d2_w'] + params['d2_b'])
        trans = (jax.nn.relu(g @ params['t1_w'] + params['t1_b'])
                 @ params['t2_w'] + params['t2_b'])
        return dos, trans
    src = edge_index[0].astype(jnp.int32)
    dst = edge_index[1].astype(jnp.int32)
    x8 = jnp.pad(x, ((0, 0), (0, 4)))
    ea8 = jnp.pad(edge_attr, ((0, 0), (0, 3)))
    npw8 = jnp.pad(params['np_w'], ((0, 4), (0, 0)))
    epw8 = jnp.pad(params['ep_w'], ((0, 3), (0, 0)))
    zrow = jnp.zeros((RPT, D), F32)
    bb8 = jnp.broadcast_to(batch.astype(F32)[:, None], (N, 8))

    h = _dense0(x8, npw8, _row(params['np_b']))
    for lp in params['layers']:
        w3 = jnp.concatenate([lp['wq'], lp['wk'], lp['wv']], axis=1)
        qt, kvt = _qkv(h, w3, _row(lp['bq']), _row(lp['bk']), _row(lp['bv']),
                       lp['we'], _row(params['ep_b']), _row(lp['be']))
        ee = _ee(ea8, epw8, lp['we'])
        gq, gkv = _sc_gather(qt, kvt, dst, src)
        wv, pb = _alpha(gq, gkv, ee)
        op, sp = _sc_scatter(wv, pb, dst, zrow)
        h = _epi(op, sp, h, lp['ws'], _row(lp['bs']),
                 _row(lp['ln_g']), _row(lp['ln_b']))

    dos, trans = _readout(h, bb8,
                          params['d1_w'], _row(params['d1_b']),
                          params['d2_w'], _row(params['d2_b']),
                          params['t1_w'], _row(params['t1_b']),
                          params['t2_w'], _row(params['t2_b']))
    return dos, trans


# speedup vs baseline: 29.2097x; 1.4864x over previous
import functools

import jax
import jax.numpy as jnp
import numpy as np
from jax import lax
from jax.experimental import pallas as pl
from jax.experimental.pallas import tpu as pltpu
from jax.experimental.pallas import tpu_sc as plsc

N = 10000
E = 320000
D = 128
H = 4
C = 32
NB = 16
OUT = 100
F32 = jnp.float32

NC = 2
NS = 16
NW = NC * NS
EPW = E // NW
CH = 80
NCHUNK = EPW // CH
NPAD = 10240
RPT = NPAD // NS

BN = 1000
BE = 2000

@functools.cache
def _sc_mesh():
    return plsc.VectorSubcoreMesh(core_axis_name="c", subcore_axis_name="s",
                                  num_cores=NC, num_subcores=NS)


def _sc_gather_body(qt, kvt, dst3, src3, gq, gkv,
                    idxd, idxs, qrows, kvrows,
                    semq0, semq1, semk0, semk1):
    cid = lax.axis_index("c")
    sid = lax.axis_index("s")
    wid = sid * NC + cid
    semq = (semq0, semq1)
    semk = (semk0, semk1)

    pltpu.sync_copy(dst3.at[wid], idxd)
    pltpu.sync_copy(src3.at[wid], idxs)

    def issue(ci, b):
        pltpu.async_copy(qt.at[idxd.at[ci]], qrows.at[b], semq[b])
        pltpu.async_copy(kvt.at[idxs.at[ci]], kvrows.at[b], semk[b])

    def wait(b):
        pltpu.make_async_copy(qt.at[idxd.at[0]], qrows.at[b], semq[b]).wait()
        pltpu.make_async_copy(kvt.at[idxs.at[0]], kvrows.at[b], semk[b]).wait()

    def write(ci, b):
        base = wid * EPW + ci * CH
        pltpu.sync_copy(qrows.at[b], gq.at[pl.ds(base, CH)])
        pltpu.sync_copy(kvrows.at[b], gkv.at[pl.ds(base, CH)])

    issue(0, 0)

    def pair(i, carry):
        ci = 2 * i
        issue(ci + 1, 1)
        wait(0)
        write(ci, 0)
        issue(ci + 2, 0)
        wait(1)
        write(ci + 1, 1)
        return carry

    lax.fori_loop(0, (NCHUNK - 1) // 2, pair, 0)
    wait(0)
    write(NCHUNK - 1, 0)


def _sc_gather(qt, kvt, dst3, src3):
    return pl.kernel(
        _sc_gather_body,
        out_type=(jax.ShapeDtypeStruct((E, D), F32),
                  jax.ShapeDtypeStruct((E, 2 * D), F32)),
        mesh=_sc_mesh(),
        scratch_types=[
            pltpu.VMEM((NCHUNK, CH), jnp.int32),
            pltpu.VMEM((NCHUNK, CH), jnp.int32),
            pltpu.VMEM((2, CH, D), F32),
            pltpu.VMEM((2, CH, 2 * D), F32),
            pltpu.SemaphoreType.DMA,
            pltpu.SemaphoreType.DMA,
            pltpu.SemaphoreType.DMA,
            pltpu.SemaphoreType.DMA,
        ],
    )(qt, kvt, dst3, src3)


def _sc_scatter_body(wv_h, pb_h, dst3, zrow_h, op_h, sp_h,
                     idx, rows, acc, sem0, sem1):
    cid = lax.axis_index("c")
    sid = lax.axis_index("s")
    wid = sid * NC + cid
    sem = (sem0, sem1)

    pltpu.sync_copy(dst3.at[wid], idx)

    for src_h, out_h in ((wv_h, op_h), (pb_h, sp_h)):
        pltpu.sync_copy(zrow_h, acc.at[pl.ds(sid * RPT, RPT)])
        plsc.subcore_barrier()

        def issue(ci, b, src_h=src_h):
            base = wid * EPW + ci * CH
            pltpu.async_copy(src_h.at[pl.ds(base, CH)], rows.at[b], sem[b])

        def wait(b, src_h=src_h):
            pltpu.make_async_copy(src_h.at[pl.ds(0, CH)], rows.at[b],
                                  sem[b]).wait()

        def scat(ci, b):
            pltpu.sync_copy(rows.at[b], acc.at[idx.at[ci]], add=True)

        issue(0, 0)

        def pair(i, carry):
            ci = 2 * i
            issue(ci + 1, 1)
            wait(0)
            scat(ci, 0)
            issue(ci + 2, 0)
            wait(1)
            scat(ci + 1, 1)
            return carry

        lax.fori_loop(0, (NCHUNK - 1) // 2, pair, 0)
        wait(0)
        scat(NCHUNK - 1, 0)

        plsc.subcore_barrier()
        pltpu.sync_copy(acc.at[pl.ds(sid * RPT, RPT)],
                        out_h.at[cid, pl.ds(sid * RPT, RPT)])
        plsc.subcore_barrier()


def _sc_scatter(wv, pb, dst3, zrow):
    return pl.kernel(
        _sc_scatter_body,
        out_type=(jax.ShapeDtypeStruct((NC, NPAD, D), F32),
                  jax.ShapeDtypeStruct((NC, NPAD, D), F32)),
        mesh=_sc_mesh(),
        scratch_types=[
            pltpu.VMEM((NCHUNK, CH), jnp.int32),
            pltpu.VMEM((2, CH, D), F32),
            pltpu.VMEM_SHARED((NPAD, D), F32),
            pltpu.SemaphoreType.DMA,
            pltpu.SemaphoreType.DMA,
        ],
    )(wv, pb, dst3, zrow)


def _head_mats(dtype=F32):
    ci = lax.broadcasted_iota(jnp.int32, (D, 8), 0)
    hi = lax.broadcasted_iota(jnp.int32, (D, 8), 1)
    hd = jnp.where((hi < H) & (ci // C == hi), 1.0, 0.0).astype(dtype)
    hi2 = lax.broadcasted_iota(jnp.int32, (8, D), 0)
    ci2 = lax.broadcasted_iota(jnp.int32, (8, D), 1)
    hx = jnp.where((hi2 < H) & (ci2 // C == hi2), 1.0, 0.0).astype(dtype)
    return hd, hx


def _dense0_body(x_ref, w_ref, b_ref, h_ref):
    h_ref[...] = (jnp.dot(x_ref[...], w_ref[...],
                          preferred_element_type=F32) + b_ref[...])


def _dense0(x8, npw8, npb):
    return pl.pallas_call(
        _dense0_body,
        grid=(N // BN,),
        in_specs=[
            pl.BlockSpec((BN, 8), lambda i: (i, 0)),
            pl.BlockSpec((8, D), lambda i: (0, 0)),
            pl.BlockSpec((1, D), lambda i: (0, 0)),
        ],
        out_specs=pl.BlockSpec((BN, D), lambda i: (i, 0)),
        out_shape=jax.ShapeDtypeStruct((N, D), F32),
    )(x8, npw8, npb)


def _qkv_body(h_ref, w_ref, bq_ref, bk_ref, bv_ref, we_ref, epb_ref, be_ref,
              qt_ref, kvt_ref):
    hw = jnp.dot(h_ref[...], w_ref[...], preferred_element_type=F32)
    de = (jnp.dot(epb_ref[...], we_ref[...], preferred_element_type=F32)
          + be_ref[...])
    qt_ref[...] = hw[:, :D] + bq_ref[...]
    kvt_ref[...] = hw[:, D:] + jnp.concatenate(
        [bk_ref[...] + de, bv_ref[...] + de], axis=1)


def _qkv(h, w3, bq, bk, bv, we, epb, be):
    return pl.pallas_call(
        _qkv_body,
        grid=(N // BN,),
        in_specs=[
            pl.BlockSpec((BN, D), lambda i: (i, 0)),
            pl.BlockSpec((D, 3 * D), lambda i: (0, 0)),
            pl.BlockSpec((1, D), lambda i: (0, 0)),
            pl.BlockSpec((1, D), lambda i: (0, 0)),
            pl.BlockSpec((1, D), lambda i: (0, 0)),
            pl.BlockSpec((D, D), lambda i: (0, 0)),
            pl.BlockSpec((1, D), lambda i: (0, 0)),
            pl.BlockSpec((1, D), lambda i: (0, 0)),
        ],
        out_specs=(pl.BlockSpec((BN, D), lambda i: (i, 0)),
                   pl.BlockSpec((BN, 2 * D), lambda i: (i, 0))),
        out_shape=(jax.ShapeDtypeStruct((N, D), F32),
                   jax.ShapeDtypeStruct((N, 2 * D), F32)),
    )(h, w3, bq, bk, bv, we, epb, be)


def _alpha_body(gq_ref, gkv_ref, ea_ref, epw_ref, we_ref, wv_ref, pb_ref):
    hd, hx = _head_mats()
    ce = jnp.dot(epw_ref[...], we_ref[...], preferred_element_type=F32)
    ee = jnp.dot(ea_ref[...], ce, preferred_element_type=F32)
    kj = gkv_ref[:, :D] + ee
    vj = gkv_ref[:, D:] + ee
    prod = gq_ref[...] * kj
    alpha = jnp.dot(prod, hd, preferred_element_type=F32) * (1.0 / np.sqrt(C))
    p = jnp.exp(alpha)
    pbig = jnp.dot(p, hx, preferred_element_type=F32)
    wv_ref[...] = vj * pbig
    pb_ref[...] = pbig


def _alpha(gq, gkv, ea8, epw8, we):
    return pl.pallas_call(
        _alpha_body,
        grid=(E // BE,),
        in_specs=[
            pl.BlockSpec((BE, D), lambda i: (i, 0)),
            pl.BlockSpec((BE, 2 * D), lambda i: (i, 0)),
            pl.BlockSpec((BE, 8), lambda i: (i, 0)),
            pl.BlockSpec((8, D), lambda i: (0, 0)),
            pl.BlockSpec((D, D), lambda i: (0, 0)),
        ],
        out_specs=(pl.BlockSpec((BE, D), lambda i: (i, 0)),
                   pl.BlockSpec((BE, D), lambda i: (i, 0))),
        out_shape=(jax.ShapeDtypeStruct((E, D), F32),
                   jax.ShapeDtypeStruct((E, D), F32)),
    )(gq, gkv, ea8, epw8, we)


def _epi_body(op_ref, sp_ref, h_ref, ws_ref, bs_ref, g_ref, b_ref, hn_ref):
    o = op_ref[0] + op_ref[1]
    sb = sp_ref[0] + sp_ref[1]
    out = o / (sb + 1e-16)
    out = out + jnp.dot(h_ref[...], ws_ref[...],
                        preferred_element_type=F32) + bs_ref[...]
    mu = jnp.mean(out, axis=1, keepdims=True)
    var = jnp.mean((out - mu) ** 2, axis=1, keepdims=True)
    out = (out - mu) * lax.rsqrt(var + 1e-5) * g_ref[...] + b_ref[...]
    hn_ref[...] = jnp.maximum(out, 0.0)


def _epi(op, sp, h, ws, bs, g, b):
    return pl.pallas_call(
        _epi_body,
        grid=(N // BN,),
        in_specs=[
            pl.BlockSpec((NC, BN, D), lambda i: (0, i, 0)),
            pl.BlockSpec((NC, BN, D), lambda i: (0, i, 0)),
            pl.BlockSpec((BN, D), lambda i: (i, 0)),
            pl.BlockSpec((D, D), lambda i: (0, 0)),
            pl.BlockSpec((1, D), lambda i: (0, 0)),
            pl.BlockSpec((1, D), lambda i: (0, 0)),
            pl.BlockSpec((1, D), lambda i: (0, 0)),
        ],
        out_specs=pl.BlockSpec((BN, D), lambda i: (i, 0)),
        out_shape=jax.ShapeDtypeStruct((N, D), F32),
    )(op, sp, h, ws, bs, g, b)


def _readout_body(h_ref, bb_ref, d1_ref, db1_ref, d2_ref, db2_ref,
                  t1_ref, tb1_ref, t2_ref, tb2_ref,
                  dos_ref, trans_ref, sums_ref, cnt_ref):
    i = pl.program_id(0)

    @pl.when(i == 0)
    def _init():
        sums_ref[...] = jnp.zeros_like(sums_ref)
        cnt_ref[...] = jnp.zeros_like(cnt_ref)

    hh = h_ref[...]
    bb = bb_ref[...][:, 0:1]
    ids = lax.broadcasted_iota(jnp.int32, (BN, NB), 1).astype(F32)
    onehot = jnp.where(jnp.broadcast_to(bb, (BN, NB)) == ids, 1.0, 0.0)
    dn = (((0,), (0,)), ((), ()))
    sums_ref[...] += lax.dot_general(onehot, hh, dn,
                                     preferred_element_type=F32)
    cnt_ref[...] += lax.dot_general(onehot, jnp.ones_like(hh), dn,
                                    preferred_element_type=F32)

    @pl.when(i == (N // BN) - 1)
    def _fin():
        g = sums_ref[...] / jnp.maximum(cnt_ref[...], 1.0)
        dd = jnp.maximum(
            jnp.dot(g, d1_ref[...], preferred_element_type=F32)
            + db1_ref[...], 0.0)
        dos_ref[...] = (jnp.dot(dd, d2_ref[...], preferred_element_type=F32)
                        + db2_ref[...])
        tt = jnp.maximum(
            jnp.dot(g, t1_ref[...], preferred_element_type=F32)
            + tb1_ref[...], 0.0)
        trans_ref[...] = (jnp.dot(tt, t2_ref[...], preferred_element_type=F32)
                          + tb2_ref[...])


def _readout(h, bb8, d1, db1, d2, db2, t1, tb1, t2, tb2):
    D2 = D // 2
    return pl.pallas_call(
        _readout_body,
        grid=(N // BN,),
        in_specs=[
            pl.BlockSpec((BN, D), lambda i: (i, 0)),
            pl.BlockSpec((BN, 8), lambda i: (i, 0)),
            pl.BlockSpec((D, D2), lambda i: (0, 0)),
            pl.BlockSpec((1, D2), lambda i: (0, 0)),
            pl.BlockSpec((D2, OUT), lambda i: (0, 0)),
            pl.BlockSpec((1, OUT), lambda i: (0, 0)),
            pl.BlockSpec((D, D2), lambda i: (0, 0)),
            pl.BlockSpec((1, D2), lambda i: (0, 0)),
            pl.BlockSpec((D2, OUT), lambda i: (0, 0)),
            pl.BlockSpec((1, OUT), lambda i: (0, 0)),
        ],
        out_specs=(pl.BlockSpec((NB, OUT), lambda i: (0, 0)),
                   pl.BlockSpec((NB, OUT), lambda i: (0, 0))),
        out_shape=(jax.ShapeDtypeStruct((NB, OUT), F32),
                   jax.ShapeDtypeStruct((NB, OUT), F32)),
        scratch_shapes=[pltpu.VMEM((NB, D), F32), pltpu.VMEM((NB, D), F32)],
    )(h, bb8, d1, db1, d2, db2, t1, tb1, t2, tb2)


def _row(v):
    return v.reshape(1, -1)


def kernel(x, edge_index, edge_attr, batch, params):
    src3 = edge_index[0].astype(jnp.int32).reshape(NW, NCHUNK, CH)
    dst3 = edge_index[1].astype(jnp.int32).reshape(NW, NCHUNK, CH)
    x8 = jnp.pad(x, ((0, 0), (0, 4)))
    ea8 = jnp.pad(edge_attr, ((0, 0), (0, 3)))
    npw8 = jnp.pad(params['np_w'], ((0, 4), (0, 0)))
    epw8 = jnp.pad(params['ep_w'], ((0, 3), (0, 0)))
    zrow = jnp.zeros((RPT, D), F32)
    bb8 = jnp.broadcast_to(batch.astype(F32)[:, None], (N, 8))

    h = _dense0(x8, npw8, _row(params['np_b']))
    for lp in params['layers']:
        w3 = jnp.concatenate([lp['wq'], lp['wk'], lp['wv']], axis=1)
        qt, kvt = _qkv(h, w3, _row(lp['bq']), _row(lp['bk']), _row(lp['bv']),
                       lp['we'], _row(params['ep_b']), _row(lp['be']))
        gq, gkv = _sc_gather(qt, kvt, dst3, src3)
        wv, pb = _alpha(gq, gkv, ea8, epw8, lp['we'])
        op, sp = _sc_scatter(wv, pb, dst3, zrow)
        h = _epi(op, sp, h, lp['ws'], _row(lp['bs']),
                 _row(lp['ln_g']), _row(lp['ln_b']))

    dos, trans = _readout(h, bb8,
                          params['d1_w'], _row(params['d1_b']),
                          params['d2_w'], _row(params['d2_b']),
                          params['t1_w'], _row(params['t1_b']),
                          params['t2_w'], _row(params['t2_b']))
    return dos, trans


# speedup vs baseline: 29.2735x; 1.0022x over previous
import functools

import jax
import jax.numpy as jnp
import numpy as np
from jax import lax
from jax.experimental import pallas as pl
from jax.experimental.pallas import tpu as pltpu
from jax.experimental.pallas import tpu_sc as plsc

N = 10000
E = 320000
D = 128
H = 4
C = 32
NB = 16
OUT = 100
F32 = jnp.float32

NC = 2
NS = 16
NW = NC * NS
EPW = E // NW
CH = 80
NCHUNK = EPW // CH
NPAD = 10240
RPT = NPAD // NS

BN = 1000
BE = 2000

@functools.cache
def _sc_mesh():
    return plsc.VectorSubcoreMesh(core_axis_name="c", subcore_axis_name="s",
                                  num_cores=NC, num_subcores=NS)


def _sc_gather_body(qt, kvt, dst3, src3, gq, gkv,
                    idxd, idxs, qrows, kvrows,
                    semq0, semq1, semq2, semk0, semk1, semk2):
    cid = lax.axis_index("c")
    sid = lax.axis_index("s")
    wid = sid * NC + cid
    semq = (semq0, semq1, semq2)
    semk = (semk0, semk1, semk2)

    pltpu.sync_copy(dst3.at[wid], idxd)
    pltpu.sync_copy(src3.at[wid], idxs)

    def issue_gather(ci, b):
        pltpu.async_copy(qt.at[idxd.at[ci]], qrows.at[b], semq[b])
        pltpu.async_copy(kvt.at[idxs.at[ci]], kvrows.at[b], semk[b])

    def wait_gather(b):
        pltpu.make_async_copy(qt.at[idxd.at[0]], qrows.at[b], semq[b]).wait()
        pltpu.make_async_copy(kvt.at[idxs.at[0]], kvrows.at[b], semk[b]).wait()

    def issue_write(ci, b):
        base = wid * EPW + ci * CH
        pltpu.async_copy(qrows.at[b], gq.at[pl.ds(base, CH)], semq[b])
        pltpu.async_copy(kvrows.at[b], gkv.at[pl.ds(base, CH)], semk[b])

    def drain_write(b):
        pltpu.make_async_copy(qrows.at[b], gq.at[pl.ds(0, CH)], semq[b]).wait()
        pltpu.make_async_copy(kvrows.at[b], gkv.at[pl.ds(0, CH)],
                              semk[b]).wait()

    def step(ci, b, first=False, last=False):
        if not first:
            drain_write((b + 2) % 3)
        if not last:
            issue_gather(ci + 2, (b + 2) % 3)
        wait_gather(b)
        issue_write(ci, b)

    issue_gather(0, 0)
    issue_gather(1, 1)
    step(0, 0, first=True)
    step(1, 1)
    step(2, 2)

    def tri(t, carry):
        ci = 3 * t
        step(ci, 0)
        step(ci + 1, 1)
        step(ci + 2, 2)
        return carry

    lax.fori_loop(1, (NCHUNK - 2) // 3, tri, 0)
    step(NCHUNK - 2, (NCHUNK - 2) % 3, last=True)
    step(NCHUNK - 1, (NCHUNK - 1) % 3, last=True)
    drain_write((NCHUNK - 1) % 3)


def _sc_gather(qt, kvt, dst3, src3):
    return pl.kernel(
        _sc_gather_body,
        out_type=(jax.ShapeDtypeStruct((E, D), F32),
                  jax.ShapeDtypeStruct((E, 2 * D), F32)),
        mesh=_sc_mesh(),
        scratch_types=[
            pltpu.VMEM((NCHUNK, CH), jnp.int32),
            pltpu.VMEM((NCHUNK, CH), jnp.int32),
            pltpu.VMEM((3, CH, D), F32),
            pltpu.VMEM((3, CH, 2 * D), F32),
            pltpu.SemaphoreType.DMA,
            pltpu.SemaphoreType.DMA,
            pltpu.SemaphoreType.DMA,
            pltpu.SemaphoreType.DMA,
            pltpu.SemaphoreType.DMA,
            pltpu.SemaphoreType.DMA,
        ],
    )(qt, kvt, dst3, src3)


def _sc_scatter_body(wv_h, pb_h, dst3, zrow_h, op_h, sp_h,
                     idx, rows, acc, sem0, sem1):
    cid = lax.axis_index("c")
    sid = lax.axis_index("s")
    wid = sid * NC + cid
    sem = (sem0, sem1)

    pltpu.sync_copy(dst3.at[wid], idx)

    for src_h, out_h in ((wv_h, op_h), (pb_h, sp_h)):
        pltpu.sync_copy(zrow_h, acc.at[pl.ds(sid * RPT, RPT)])
        plsc.subcore_barrier()

        def issue(ci, b, src_h=src_h):
            base = wid * EPW + ci * CH
            pltpu.async_copy(src_h.at[pl.ds(base, CH)], rows.at[b], sem[b])

        def wait(b, src_h=src_h):
            pltpu.make_async_copy(src_h.at[pl.ds(0, CH)], rows.at[b],
                                  sem[b]).wait()

        def scat(ci, b):
            pltpu.sync_copy(rows.at[b], acc.at[idx.at[ci]], add=True)

        issue(0, 0)

        def pair(i, carry):
            ci = 2 * i
            issue(ci + 1, 1)
            wait(0)
            scat(ci, 0)
            issue(ci + 2, 0)
            wait(1)
            scat(ci + 1, 1)
            return carry

        lax.fori_loop(0, (NCHUNK - 1) // 2, pair, 0)
        wait(0)
        scat(NCHUNK - 1, 0)

        plsc.subcore_barrier()
        pltpu.sync_copy(acc.at[pl.ds(sid * RPT, RPT)],
                        out_h.at[cid, pl.ds(sid * RPT, RPT)])
        plsc.subcore_barrier()


def _sc_scatter(wv, pb, dst3, zrow):
    return pl.kernel(
        _sc_scatter_body,
        out_type=(jax.ShapeDtypeStruct((NC, NPAD, D), F32),
                  jax.ShapeDtypeStruct((NC, NPAD, D), F32)),
        mesh=_sc_mesh(),
        scratch_types=[
            pltpu.VMEM((NCHUNK, CH), jnp.int32),
            pltpu.VMEM((2, CH, D), F32),
            pltpu.VMEM_SHARED((NPAD, D), F32),
            pltpu.SemaphoreType.DMA,
            pltpu.SemaphoreType.DMA,
        ],
    )(wv, pb, dst3, zrow)


def _head_mats(dtype=F32):
    ci = lax.broadcasted_iota(jnp.int32, (D, 8), 0)
    hi = lax.broadcasted_iota(jnp.int32, (D, 8), 1)
    hd = jnp.where((hi < H) & (ci // C == hi), 1.0, 0.0).astype(dtype)
    hi2 = lax.broadcasted_iota(jnp.int32, (8, D), 0)
    ci2 = lax.broadcasted_iota(jnp.int32, (8, D), 1)
    hx = jnp.where((hi2 < H) & (ci2 // C == hi2), 1.0, 0.0).astype(dtype)
    return hd, hx


def _dense0_body(x_ref, w_ref, b_ref, h_ref):
    h_ref[...] = (jnp.dot(x_ref[...], w_ref[...],
                          preferred_element_type=F32) + b_ref[...])


def _dense0(x8, npw8, npb):
    return pl.pallas_call(
        _dense0_body,
        grid=(N // BN,),
        in_specs=[
            pl.BlockSpec((BN, 8), lambda i: (i, 0)),
            pl.BlockSpec((8, D), lambda i: (0, 0)),
            pl.BlockSpec((1, D), lambda i: (0, 0)),
        ],
        out_specs=pl.BlockSpec((BN, D), lambda i: (i, 0)),
        out_shape=jax.ShapeDtypeStruct((N, D), F32),
    )(x8, npw8, npb)


def _qkv_body(h_ref, w_ref, bq_ref, bk_ref, bv_ref, we_ref, epb_ref, be_ref,
              qt_ref, kvt_ref):
    hw = jnp.dot(h_ref[...], w_ref[...], preferred_element_type=F32)
    de = (jnp.dot(epb_ref[...], we_ref[...], preferred_element_type=F32)
          + be_ref[...])
    qt_ref[...] = hw[:, :D] + bq_ref[...]
    kvt_ref[...] = hw[:, D:] + jnp.concatenate(
        [bk_ref[...] + de, bv_ref[...] + de], axis=1)


def _qkv(h, w3, bq, bk, bv, we, epb, be):
    return pl.pallas_call(
        _qkv_body,
        grid=(N // BN,),
        in_specs=[
            pl.BlockSpec((BN, D), lambda i: (i, 0)),
            pl.BlockSpec((D, 3 * D), lambda i: (0, 0)),
            pl.BlockSpec((1, D), lambda i: (0, 0)),
            pl.BlockSpec((1, D), lambda i: (0, 0)),
            pl.BlockSpec((1, D), lambda i: (0, 0)),
            pl.BlockSpec((D, D), lambda i: (0, 0)),
            pl.BlockSpec((1, D), lambda i: (0, 0)),
            pl.BlockSpec((1, D), lambda i: (0, 0)),
        ],
        out_specs=(pl.BlockSpec((BN, D), lambda i: (i, 0)),
                   pl.BlockSpec((BN, 2 * D), lambda i: (i, 0))),
        out_shape=(jax.ShapeDtypeStruct((N, D), F32),
                   jax.ShapeDtypeStruct((N, 2 * D), F32)),
    )(h, w3, bq, bk, bv, we, epb, be)


def _alpha_body(gq_ref, gkv_ref, ea_ref, epw_ref, we_ref, wv_ref, pb_ref):
    hd, hx = _head_mats()
    ce = jnp.dot(epw_ref[...], we_ref[...], preferred_element_type=F32)
    ee = jnp.dot(ea_ref[...], ce, preferred_element_type=F32)
    kj = gkv_ref[:, :D] + ee
    vj = gkv_ref[:, D:] + ee
    prod = gq_ref[...] * kj
    alpha = jnp.dot(prod, hd, preferred_element_type=F32) * (1.0 / np.sqrt(C))
    p = jnp.exp(alpha)
    pbig = jnp.dot(p, hx, preferred_element_type=F32)
    wv_ref[...] = vj * pbig
    pb_ref[...] = pbig


def _alpha(gq, gkv, ea8, epw8, we):
    return pl.pallas_call(
        _alpha_body,
        grid=(E // BE,),
        in_specs=[
            pl.BlockSpec((BE, D), lambda i: (i, 0)),
            pl.BlockSpec((BE, 2 * D), lambda i: (i, 0)),
            pl.BlockSpec((BE, 8), lambda i: (i, 0)),
            pl.BlockSpec((8, D), lambda i: (0, 0)),
            pl.BlockSpec((D, D), lambda i: (0, 0)),
        ],
        out_specs=(pl.BlockSpec((BE, D), lambda i: (i, 0)),
                   pl.BlockSpec((BE, D), lambda i: (i, 0))),
        out_shape=(jax.ShapeDtypeStruct((E, D), F32),
                   jax.ShapeDtypeStruct((E, D), F32)),
    )(gq, gkv, ea8, epw8, we)


def _epi_body(op_ref, sp_ref, h_ref, ws_ref, bs_ref, g_ref, b_ref, hn_ref):
    o = op_ref[0] + op_ref[1]
    sb = sp_ref[0] + sp_ref[1]
    out = o / (sb + 1e-16)
    out = out + jnp.dot(h_ref[...], ws_ref[...],
                        preferred_element_type=F32) + bs_ref[...]
    mu = jnp.mean(out, axis=1, keepdims=True)
    var = jnp.mean((out - mu) ** 2, axis=1, keepdims=True)
    out = (out - mu) * lax.rsqrt(var + 1e-5) * g_ref[...] + b_ref[...]
    hn_ref[...] = jnp.maximum(out, 0.0)


def _epi(op, sp, h, ws, bs, g, b):
    return pl.pallas_call(
        _epi_body,
        grid=(N // BN,),
        in_specs=[
            pl.BlockSpec((NC, BN, D), lambda i: (0, i, 0)),
            pl.BlockSpec((NC, BN, D), lambda i: (0, i, 0)),
            pl.BlockSpec((BN, D), lambda i: (i, 0)),
            pl.BlockSpec((D, D), lambda i: (0, 0)),
            pl.BlockSpec((1, D), lambda i: (0, 0)),
            pl.BlockSpec((1, D), lambda i: (0, 0)),
            pl.BlockSpec((1, D), lambda i: (0, 0)),
        ],
        out_specs=pl.BlockSpec((BN, D), lambda i: (i, 0)),
        out_shape=jax.ShapeDtypeStruct((N, D), F32),
    )(op, sp, h, ws, bs, g, b)


def _readout_body(h_ref, bb_ref, d1_ref, db1_ref, d2_ref, db2_ref,
                  t1_ref, tb1_ref, t2_ref, tb2_ref,
                  dos_ref, trans_ref, sums_ref, cnt_ref):
    i = pl.program_id(0)

    @pl.when(i == 0)
    def _init():
        sums_ref[...] = jnp.zeros_like(sums_ref)
        cnt_ref[...] = jnp.zeros_like(cnt_ref)

    hh = h_ref[...]
    bb = bb_ref[...][:, 0:1]
    ids = lax.broadcasted_iota(jnp.int32, (BN, NB), 1).astype(F32)
    onehot = jnp.where(jnp.broadcast_to(bb, (BN, NB)) == ids, 1.0, 0.0)
    dn = (((0,), (0,)), ((), ()))
    sums_ref[...] += lax.dot_general(onehot, hh, dn,
                                     preferred_element_type=F32)
    cnt_ref[...] += lax.dot_general(onehot, jnp.ones_like(hh), dn,
                                    preferred_element_type=F32)

    @pl.when(i == (N // BN) - 1)
    def _fin():
        g = sums_ref[...] / jnp.maximum(cnt_ref[...], 1.0)
        dd = jnp.maximum(
            jnp.dot(g, d1_ref[...], preferred_element_type=F32)
            + db1_ref[...], 0.0)
        dos_ref[...] = (jnp.dot(dd, d2_ref[...], preferred_element_type=F32)
                        + db2_ref[...])
        tt = jnp.maximum(
            jnp.dot(g, t1_ref[...], preferred_element_type=F32)
            + tb1_ref[...], 0.0)
        trans_ref[...] = (jnp.dot(tt, t2_ref[...], preferred_element_type=F32)
                          + tb2_ref[...])


def _readout(h, bb8, d1, db1, d2, db2, t1, tb1, t2, tb2):
    D2 = D // 2
    return pl.pallas_call(
        _readout_body,
        grid=(N // BN,),
        in_specs=[
            pl.BlockSpec((BN, D), lambda i: (i, 0)),
            pl.BlockSpec((BN, 8), lambda i: (i, 0)),
            pl.BlockSpec((D, D2), lambda i: (0, 0)),
            pl.BlockSpec((1, D2), lambda i: (0, 0)),
            pl.BlockSpec((D2, OUT), lambda i: (0, 0)),
            pl.BlockSpec((1, OUT), lambda i: (0, 0)),
            pl.BlockSpec((D, D2), lambda i: (0, 0)),
            pl.BlockSpec((1, D2), lambda i: (0, 0)),
            pl.BlockSpec((D2, OUT), lambda i: (0, 0)),
            pl.BlockSpec((1, OUT), lambda i: (0, 0)),
        ],
        out_specs=(pl.BlockSpec((NB, OUT), lambda i: (0, 0)),
                   pl.BlockSpec((NB, OUT), lambda i: (0, 0))),
        out_shape=(jax.ShapeDtypeStruct((NB, OUT), F32),
                   jax.ShapeDtypeStruct((NB, OUT), F32)),
        scratch_shapes=[pltpu.VMEM((NB, D), F32), pltpu.VMEM((NB, D), F32)],
    )(h, bb8, d1, db1, d2, db2, t1, tb1, t2, tb2)


def _row(v):
    return v.reshape(1, -1)


def kernel(x, edge_index, edge_attr, batch, params):
    src3 = edge_index[0].astype(jnp.int32).reshape(NW, NCHUNK, CH)
    dst3 = edge_index[1].astype(jnp.int32).reshape(NW, NCHUNK, CH)
    x8 = jnp.pad(x, ((0, 0), (0, 4)))
    ea8 = jnp.pad(edge_attr, ((0, 0), (0, 3)))
    npw8 = jnp.pad(params['np_w'], ((0, 4), (0, 0)))
    epw8 = jnp.pad(params['ep_w'], ((0, 3), (0, 0)))
    zrow = jnp.zeros((RPT, D), F32)
    bb8 = jnp.broadcast_to(batch.astype(F32)[:, None], (N, 8))

    h = _dense0(x8, npw8, _row(params['np_b']))
    for lp in params['layers']:
        w3 = jnp.concatenate([lp['wq'], lp['wk'], lp['wv']], axis=1)
        qt, kvt = _qkv(h, w3, _row(lp['bq']), _row(lp['bk']), _row(lp['bv']),
                       lp['we'], _row(params['ep_b']), _row(lp['be']))
        gq, gkv = _sc_gather(qt, kvt, dst3, src3)
        wv, pb = _alpha(gq, gkv, ea8, epw8, lp['we'])
        op, sp = _sc_scatter(wv, pb, dst3, zrow)
        h = _epi(op, sp, h, lp['ws'], _row(lp['bs']),
                 _row(lp['ln_g']), _row(lp['ln_b']))

    dos, trans = _readout(h, bb8,
                          params['d1_w'], _row(params['d1_b']),
                          params['d2_w'], _row(params['d2_b']),
                          params['t1_w'], _row(params['t1_b']),
                          params['t2_w'], _row(params['t2_b']))
    return dos, trans
